# Initial kernel scaffold; baseline (speedup 1.0000x reference)
import jax
import jax.numpy as jnp
from jax.experimental import pallas as pl


def kernel(hidden_states, dif_granularity_features, text_hidden_states, text_attention_mask, router_w1, router_b1, router_w2, router_b2, voter, norm_g, norm_b, in_proj_w, in_proj_b, out_proj_w, out_proj_b, lin1_w, lin1_b, lin2_w, lin2_b, norm1_g, norm1_b, norm2_g, norm2_b):
    raise NotImplementedError("write your pallas kernel here")


# speedup vs baseline: 1.9791x; 1.9791x over previous
import functools

import jax
import jax.numpy as jnp
import numpy as np
from jax.experimental import pallas as pl
from jax.experimental.pallas import tpu as pltpu

D = 1024
E = 4
L = 576
NIMG = 4
T_TXT = 64
TQ = 32
NUM_TOKENS = E * L + TQ
DFF = 1024
H = 8
DH = D // H
TILE_L = 144
NLB = L // TILE_L

_F32 = jnp.float32
_BF16 = jnp.bfloat16


def _ln(x, g, b, eps=1e-5):
    mu = x.mean(-1, keepdims=True)
    var = ((x - mu) ** 2).mean(-1, keepdims=True)
    return (x - mu) * jax.lax.rsqrt(var + eps) * g + b


def _gelu(x):
    return 0.5 * x * (1.0 + jax.lax.erf(x * np.float32(1.0 / np.sqrt(2.0))))


def _dot(a, b):
    return jnp.dot(a, b, preferred_element_type=_F32,
                   precision=jax.lax.Precision.HIGHEST)


def _r(a):
    return a.astype(_BF16).astype(_F32)


def _dotb(a, b):
    return jnp.dot(a.astype(_BF16), b, preferred_element_type=_F32)


def _text_kernel(img_ref, txt_ref, mrow_ref, vt_ref,
                 wv_ref, bv_ref, wo_ref, bo_ref, l1_ref, b1_ref, l2_ref, b2_ref,
                 rw1_ref, rb1_ref, rw2_ref,
                 n1g_ref, n1b_ref, n2g_ref, n2b_ref, out_ref):
    img = img_ref[:]
    txt = txt_ref[:]
    nrm = jnp.sqrt((img * img).sum(-1, keepdims=True))
    xn = img / jnp.maximum(nrm, 1e-8)
    tnr = jnp.sqrt((txt * txt).sum(-1, keepdims=True))
    tn = txt / jnp.maximum(tnr, 1e-8)
    sim = jax.lax.dot_general(xn, tn, (((1,), (1,)), ((), ())),
                              preferred_element_type=_F32)
    neg = np.float32(-1e30)
    sim = jnp.where(mrow_ref[:] == 0.0, neg, sim)
    a_row = -sim.mean(0, keepdims=True)
    a_col = jnp.transpose(a_row)
    AR = jnp.broadcast_to(a_row, (T_TXT, T_TXT))
    AC = jnp.broadcast_to(a_col, (T_TXT, T_TXT))
    ii = jax.lax.broadcasted_iota(jnp.int32, (T_TXT, T_TXT), 0)
    jj = jax.lax.broadcasted_iota(jnp.int32, (T_TXT, T_TXT), 1)
    before = (AC < AR) | ((AC == AR) & (ii < jj))
    rank_row = before.astype(_F32).sum(0, keepdims=True)
    rr = jnp.broadcast_to(rank_row, (TQ, T_TXT))
    si = jax.lax.broadcasted_iota(jnp.int32, (TQ, T_TXT), 0).astype(_F32)
    P = (rr == si).astype(_F32)
    new_text = _dot(P, txt)
    v = _dotb(new_text, wv_ref[:]) + bv_ref[:]
    ao = _dotb(v, wo_ref[:]) + bo_ref[:]
    x1 = _ln(new_text + ao, n1g_ref[:], n1b_ref[:])
    ffh = jnp.maximum(_dotb(x1, l1_ref[:]) + b1_ref[:], 0.0)
    ff = _dotb(ffh, l2_ref[:]) + b2_ref[:]
    xo = _ln(x1 + ff, n2g_ref[:], n2b_ref[:])
    h = _gelu(_dotb(xo, rw1_ref[:]) + rb1_ref[:])
    lt = _dotb(h, rw2_ref[:])
    out_ref[:] = (_r(vt_ref[:]) * _r(lt)).sum(0, keepdims=True)


def _feat_kernel(dif_ref, voter_ref, hm_ref, hmt_ref,
                 wq_ref, wk_ref, wv_ref, wo_ref, l1_ref, l2_ref, rw1_ref,
                 rw2_ref,
                 bq_ref, bk_ref, bv_ref, bo_ref, b1_ref, b2_ref, rb1_ref,
                 n1g_ref, n1b_ref, n2g_ref, n2b_ref, out_ref):
    e = pl.program_id(0)
    lb = pl.program_id(1)
    R = NIMG * TILE_L
    x = dif_ref[0].reshape(R, D)
    iscale = np.float32(1.0 / np.sqrt(DH))
    q = _dotb(x, wq_ref[:]) + bq_ref[:]
    k = _dotb(x, wk_ref[:]) + bk_ref[:]
    v = _dotb(x, wv_ref[:]) + bv_ref[:]
    qs = [_r(q[s * TILE_L:(s + 1) * TILE_L]) for s in range(NIMG)]
    ks = [_r(k[t * TILE_L:(t + 1) * TILE_L]) for t in range(NIMG)]
    vs = [_r(v[t * TILE_L:(t + 1) * TILE_L]) for t in range(NIMG)]
    attn_parts = []
    for s in range(NIMG):
        sc = [_dot(qs[s] * ks[t], hm_ref[:]) * iscale
              for t in range(NIMG)]
        mx = jnp.maximum(jnp.maximum(sc[0], sc[1]), jnp.maximum(sc[2], sc[3]))
        ex = [jnp.exp(c - mx) for c in sc]
        den = ex[0] + ex[1] + ex[2] + ex[3]
        acc = None
        for t in range(NIMG):
            w = (ex[t] / den).astype(_BF16)
            wx = jnp.dot(w, hmt_ref[:], preferred_element_type=_F32)
            acc = wx * vs[t] if acc is None else acc + wx * vs[t]
        attn_parts.append(acc)
    attn = jnp.concatenate(attn_parts, axis=0)
    ao = _dotb(attn, wo_ref[:]) + bo_ref[:]
    x1 = _ln(x + ao, n1g_ref[:], n1b_ref[:])
    ffh = jnp.maximum(_dotb(x1, l1_ref[:]) + b1_ref[:], 0.0)
    ff = _dotb(ffh, l2_ref[:]) + b2_ref[:]
    xo = _ln(x1 + ff, n2g_ref[:], n2b_ref[:])
    h = _gelu(_dotb(xo, rw1_ref[:]) + rb1_ref[:])
    lt = _dotb(h, rw2_ref[:])
    off = e * L + lb * TILE_L
    w = voter_ref[pl.ds(off, TILE_L), :]
    wfull = _r(jnp.concatenate([w, w, w, w], axis=0))
    contrib = (wfull * _r(lt)).sum(0, keepdims=True)
    first = jnp.logical_and(e == 0, lb == 0)

    @pl.when(first)
    def _():
        out_ref[:] = contrib

    @pl.when(jnp.logical_not(first))
    def _():
        out_ref[:] = out_ref[:] + contrib


def _final_kernel(sf_ref, st_ref, rb2_ref, ng_ref, nb_ref, voter_ref,
                  logits_ref, idx_ref):
    vsum = voter_ref[:].sum()
    lg = 0.25 * sf_ref[:] + st_ref[:] + vsum * rb2_ref[:]
    ln = _ln(lg, ng_ref[:], nb_ref[:])
    logits_ref[:] = ln
    m = ln.max(1, keepdims=True)
    lane = jax.lax.broadcasted_iota(jnp.int32, (1, E), 1)
    idx = jnp.where(ln == m, lane, jnp.int32(10 ** 6)).min(1, keepdims=True)
    idx_ref[:] = idx


def _gather_kernel(idx_ref, dif_ref, out_ref):
    del idx_ref
    out_ref[:] = _r(dif_ref[0])


def kernel(hidden_states, dif_granularity_features, text_hidden_states,
           text_attention_mask, router_w1, router_b1, router_w2, router_b2,
           voter, norm_g, norm_b, in_proj_w, in_proj_b, out_proj_w, out_proj_b,
           lin1_w, lin1_b, lin2_w, lin2_b, norm1_g, norm1_b, norm2_g, norm2_b):
    f32 = _F32
    bf16 = _BF16
    img = hidden_states[0]
    maskf = text_attention_mask.astype(f32)
    mrow = maskf.reshape(1, T_TXT)
    wq_t = in_proj_w[:D].T.astype(bf16)
    wk_t = in_proj_w[D:2 * D].T.astype(bf16)
    wv_t = in_proj_w[2 * D:].T.astype(bf16)
    bq = in_proj_b[:D].reshape(1, D)
    bk = in_proj_b[D:2 * D].reshape(1, D)
    bv = in_proj_b[2 * D:].reshape(1, D)
    wo_t = out_proj_w.T.astype(bf16)
    bo = out_proj_b.reshape(1, D)
    l1_t = lin1_w.T.astype(bf16)
    b1 = lin1_b.reshape(1, DFF)
    l2_t = lin2_w.T.astype(bf16)
    b2 = lin2_b.reshape(1, D)
    rw1_t = router_w1.T.astype(bf16)
    rb1 = router_b1.reshape(1, D)
    rw2_t = router_w2.T.astype(bf16)
    rb2 = router_b2.reshape(1, E)
    n1g = norm1_g.reshape(1, D)
    n1b = norm1_b.reshape(1, D)
    n2g = norm2_g.reshape(1, D)
    n2b = norm2_b.reshape(1, D)
    ng = norm_g.reshape(1, E)
    nb = norm_b.reshape(1, E)
    voter_text = voter[E * L:]

    st = pl.pallas_call(
        _text_kernel,
        out_shape=jax.ShapeDtypeStruct((1, E), f32),
    )(img, text_hidden_states, mrow, voter_text,
      wv_t, bv, wo_t, bo, l1_t, b1, l2_t, b2, rw1_t, rb1, rw2_t,
      n1g, n1b, n2g, n2b)

    hm = jnp.asarray(np.repeat(np.eye(H, dtype=np.float32), DH, axis=0))
    hmt = jnp.asarray(np.repeat(np.eye(H, dtype=np.float32), DH,
                                axis=0).T.astype(np.float32)).astype(bf16)

    full = lambda shape: pl.BlockSpec(shape, lambda e, lb: tuple(0 for _ in shape))
    sf = pl.pallas_call(
        _feat_kernel,
        grid=(E, NLB),
        in_specs=[
            pl.BlockSpec((1, NIMG, TILE_L, D), lambda e, lb: (e, 0, lb, 0)),
            full((NUM_TOKENS, 1)),
            full((D, H)), full((H, D)),
            full((D, D)), full((D, D)), full((D, D)), full((D, D)),
            full((D, DFF)), full((DFF, D)), full((D, D)), full((D, E)),
            full((1, D)), full((1, D)), full((1, D)), full((1, D)),
            full((1, DFF)), full((1, D)), full((1, D)),
            full((1, D)), full((1, D)), full((1, D)), full((1, D)),
        ],
        out_specs=pl.BlockSpec((1, E), lambda e, lb: (0, 0)),
        out_shape=jax.ShapeDtypeStruct((1, E), f32),
    )(dif_granularity_features, voter, hm, hmt,
      wq_t, wk_t, wv_t, wo_t, l1_t, l2_t, rw1_t, rw2_t,
      bq, bk, bv, bo, b1, b2, rb1,
      n1g, n1b, n2g, n2b)

    logits, idx = pl.pallas_call(
        _final_kernel,
        out_shape=(jax.ShapeDtypeStruct((1, E), f32),
                   jax.ShapeDtypeStruct((1, 1), jnp.int32)),
    )(sf, st, rb2, ng, nb, voter)

    idx_arr = idx.reshape((1,))
    final = pl.pallas_call(
        _gather_kernel,
        grid_spec=pltpu.PrefetchScalarGridSpec(
            num_scalar_prefetch=1,
            grid=(NIMG,),
            in_specs=[pl.BlockSpec((1, 1, L, D),
                                   lambda n, idx_ref: (idx_ref[0], n, 0, 0))],
            out_specs=pl.BlockSpec((1, L, D), lambda n, idx_ref: (n, 0, 0)),
        ),
        out_shape=jax.ShapeDtypeStruct((NIMG, L, D), f32),
    )(idx_arr, dif_granularity_features)

    return final, logits


# speedup vs baseline: 2.4577x; 1.2418x over previous
import functools

import jax
import jax.numpy as jnp
import numpy as np
from jax.experimental import pallas as pl
from jax.experimental.pallas import tpu as pltpu

D = 1024
E = 4
L = 576
NIMG = 4
T_TXT = 64
TQ = 32
NUM_TOKENS = E * L + TQ
DFF = 1024
H = 8
DH = D // H
TILE_L = 144
NLB = L // TILE_L

_F32 = jnp.float32
_BF16 = jnp.bfloat16


def _ln(x, g, b, eps=1e-5):
    mu = x.mean(-1, keepdims=True)
    var = ((x - mu) ** 2).mean(-1, keepdims=True)
    return (x - mu) * jax.lax.rsqrt(var + eps) * g + b


def _gelu(x):
    return 0.5 * x * (1.0 + jax.lax.erf(x * np.float32(1.0 / np.sqrt(2.0))))


def _dot(a, b):
    return jnp.dot(a, b, preferred_element_type=_F32,
                   precision=jax.lax.Precision.HIGHEST)


def _r(a):
    return a.astype(_BF16).astype(_F32)


def _dotb(a, b):
    return jnp.dot(a.astype(_BF16), b, preferred_element_type=_F32)


def _text_kernel(img_ref, txt_ref, mrow_ref, vt_ref,
                 wv_ref, bv_ref, wo_ref, bo_ref, l1_ref, b1_ref, l2_ref, b2_ref,
                 rw1_ref, rb1_ref, rw2_ref,
                 n1g_ref, n1b_ref, n2g_ref, n2b_ref, out_ref):
    img = img_ref[:]
    txt = txt_ref[:]
    nrm = jnp.sqrt((img * img).sum(-1, keepdims=True))
    xn = img / jnp.maximum(nrm, 1e-8)
    tnr = jnp.sqrt((txt * txt).sum(-1, keepdims=True))
    tn = txt / jnp.maximum(tnr, 1e-8)
    sim = jax.lax.dot_general(xn, tn, (((1,), (1,)), ((), ())),
                              preferred_element_type=_F32)
    neg = np.float32(-1e30)
    sim = jnp.where(mrow_ref[:] == 0.0, neg, sim)
    a_row = -sim.mean(0, keepdims=True)
    a_col = jnp.transpose(a_row)
    AR = jnp.broadcast_to(a_row, (T_TXT, T_TXT))
    AC = jnp.broadcast_to(a_col, (T_TXT, T_TXT))
    ii = jax.lax.broadcasted_iota(jnp.int32, (T_TXT, T_TXT), 0)
    jj = jax.lax.broadcasted_iota(jnp.int32, (T_TXT, T_TXT), 1)
    before = (AC < AR) | ((AC == AR) & (ii < jj))
    rank_row = before.astype(_F32).sum(0, keepdims=True)
    rr = jnp.broadcast_to(rank_row, (TQ, T_TXT))
    si = jax.lax.broadcasted_iota(jnp.int32, (TQ, T_TXT), 0).astype(_F32)
    P = (rr == si).astype(_F32)
    new_text = _dot(P, txt)
    v = _dotb(new_text, wv_ref[:]) + bv_ref[:]
    ao = _dotb(v, wo_ref[:]) + bo_ref[:]
    x1 = _ln(new_text + ao, n1g_ref[:], n1b_ref[:])
    ffh = jnp.maximum(_dotb(x1, l1_ref[:]) + b1_ref[:], 0.0)
    ff = _dotb(ffh, l2_ref[:]) + b2_ref[:]
    xo = _ln(x1 + ff, n2g_ref[:], n2b_ref[:])
    h = _gelu(_dotb(xo, rw1_ref[:]) + rb1_ref[:])
    lt = _dotb(h, rw2_ref[:])
    out_ref[:] = (_r(vt_ref[:]) * _r(lt)).sum(0, keepdims=True)


def _feat_kernel(dif_ref, voter_ref, hm_ref, hmt_ref,
                 wq_ref, wk_ref, wv_ref, wo_ref, l1_ref, l2_ref, rw1_ref,
                 rw2_ref,
                 bq_ref, bk_ref, bv_ref, bo_ref, b1_ref, b2_ref, rb1_ref,
                 n1g_ref, n1b_ref, n2g_ref, n2b_ref, out_ref):
    e = pl.program_id(0)
    lb = pl.program_id(1)
    R = NIMG * TILE_L
    x = dif_ref[0].reshape(R, D)
    iscale = np.float32(1.0 / np.sqrt(DH))
    q = _dotb(x, wq_ref[:]) + bq_ref[:]
    k = _dotb(x, wk_ref[:]) + bk_ref[:]
    v = _dotb(x, wv_ref[:]) + bv_ref[:]
    qs = [_r(q[s * TILE_L:(s + 1) * TILE_L]) for s in range(NIMG)]
    ks = [_r(k[t * TILE_L:(t + 1) * TILE_L]) for t in range(NIMG)]
    vs = [_r(v[t * TILE_L:(t + 1) * TILE_L]) for t in range(NIMG)]
    attn_parts = []
    for s in range(NIMG):
        sc = []
        for t in range(NIMG):
            prod = qs[s] * ks[t]
            hi = prod.astype(_BF16)
            lo = (prod - hi.astype(_F32)).astype(_BF16)
            ss = (jnp.dot(hi, hm_ref[:], preferred_element_type=_F32)
                  + jnp.dot(lo, hm_ref[:], preferred_element_type=_F32))
            sc.append(ss * iscale)
        mx = jnp.maximum(jnp.maximum(sc[0], sc[1]), jnp.maximum(sc[2], sc[3]))
        ex = [jnp.exp(c - mx) for c in sc]
        den = ex[0] + ex[1] + ex[2] + ex[3]
        acc = None
        for t in range(NIMG):
            w = (ex[t] / den).astype(_BF16)
            wx = jnp.dot(w, hmt_ref[:], preferred_element_type=_F32)
            acc = wx * vs[t] if acc is None else acc + wx * vs[t]
        attn_parts.append(acc)
    attn = jnp.concatenate(attn_parts, axis=0)
    ao = _dotb(attn, wo_ref[:]) + bo_ref[:]
    x1 = _ln(x + ao, n1g_ref[:], n1b_ref[:])
    ffh = jnp.maximum(_dotb(x1, l1_ref[:]) + b1_ref[:], 0.0)
    ff = _dotb(ffh, l2_ref[:]) + b2_ref[:]
    xo = _ln(x1 + ff, n2g_ref[:], n2b_ref[:])
    h = _gelu(_dotb(xo, rw1_ref[:]) + rb1_ref[:])
    lt = _dotb(h, rw2_ref[:])
    off = e * L + lb * TILE_L
    w = voter_ref[pl.ds(off, TILE_L), :]
    wfull = _r(jnp.concatenate([w, w, w, w], axis=0))
    contrib = (wfull * _r(lt)).sum(0, keepdims=True)
    first = jnp.logical_and(e == 0, lb == 0)

    @pl.when(first)
    def _():
        out_ref[:] = contrib

    @pl.when(jnp.logical_not(first))
    def _():
        out_ref[:] = out_ref[:] + contrib


def _final_kernel(sf_ref, st_ref, rb2_ref, ng_ref, nb_ref, voter_ref,
                  logits_ref, idx_ref):
    vsum = voter_ref[:].sum()
    lg = 0.25 * sf_ref[:] + st_ref[:] + vsum * rb2_ref[:]
    ln = _ln(lg, ng_ref[:], nb_ref[:])
    logits_ref[:] = ln
    m = ln.max(1, keepdims=True)
    lane = jax.lax.broadcasted_iota(jnp.int32, (1, E), 1)
    idx = jnp.where(ln == m, lane, jnp.int32(10 ** 6)).min(1, keepdims=True)
    idx_ref[:] = idx


def _gather_kernel(idx_ref, dif_ref, out_ref):
    del idx_ref
    out_ref[:] = _r(dif_ref[0])


def kernel(hidden_states, dif_granularity_features, text_hidden_states,
           text_attention_mask, router_w1, router_b1, router_w2, router_b2,
           voter, norm_g, norm_b, in_proj_w, in_proj_b, out_proj_w, out_proj_b,
           lin1_w, lin1_b, lin2_w, lin2_b, norm1_g, norm1_b, norm2_g, norm2_b):
    f32 = _F32
    bf16 = _BF16
    img = hidden_states[0]
    maskf = text_attention_mask.astype(f32)
    mrow = maskf.reshape(1, T_TXT)
    wq_t = in_proj_w[:D].T.astype(bf16)
    wk_t = in_proj_w[D:2 * D].T.astype(bf16)
    wv_t = in_proj_w[2 * D:].T.astype(bf16)
    bq = in_proj_b[:D].reshape(1, D)
    bk = in_proj_b[D:2 * D].reshape(1, D)
    bv = in_proj_b[2 * D:].reshape(1, D)
    wo_t = out_proj_w.T.astype(bf16)
    bo = out_proj_b.reshape(1, D)
    l1_t = lin1_w.T.astype(bf16)
    b1 = lin1_b.reshape(1, DFF)
    l2_t = lin2_w.T.astype(bf16)
    b2 = lin2_b.reshape(1, D)
    rw1_t = router_w1.T.astype(bf16)
    rb1 = router_b1.reshape(1, D)
    rw2_t = router_w2.T.astype(bf16)
    rb2 = router_b2.reshape(1, E)
    n1g = norm1_g.reshape(1, D)
    n1b = norm1_b.reshape(1, D)
    n2g = norm2_g.reshape(1, D)
    n2b = norm2_b.reshape(1, D)
    ng = norm_g.reshape(1, E)
    nb = norm_b.reshape(1, E)
    voter_text = voter[E * L:]

    st = pl.pallas_call(
        _text_kernel,
        out_shape=jax.ShapeDtypeStruct((1, E), f32),
    )(img, text_hidden_states, mrow, voter_text,
      wv_t, bv, wo_t, bo, l1_t, b1, l2_t, b2, rw1_t, rb1, rw2_t,
      n1g, n1b, n2g, n2b)

    hm = jnp.asarray(np.repeat(np.eye(H, dtype=np.float32), DH, axis=0))
    hmt = jnp.asarray(np.repeat(np.eye(H, dtype=np.float32), DH,
                                axis=0).T.astype(np.float32)).astype(bf16)

    full = lambda shape: pl.BlockSpec(shape, lambda e, lb: tuple(0 for _ in shape))
    sf = pl.pallas_call(
        _feat_kernel,
        grid=(E, NLB),
        in_specs=[
            pl.BlockSpec((1, NIMG, TILE_L, D), lambda e, lb: (e, 0, lb, 0)),
            full((NUM_TOKENS, 1)),
            full((D, H)), full((H, D)),
            full((D, D)), full((D, D)), full((D, D)), full((D, D)),
            full((D, DFF)), full((DFF, D)), full((D, D)), full((D, E)),
            full((1, D)), full((1, D)), full((1, D)), full((1, D)),
            full((1, DFF)), full((1, D)), full((1, D)),
            full((1, D)), full((1, D)), full((1, D)), full((1, D)),
        ],
        out_specs=pl.BlockSpec((1, E), lambda e, lb: (0, 0)),
        out_shape=jax.ShapeDtypeStruct((1, E), f32),
    )(dif_granularity_features, voter, hm, hmt,
      wq_t, wk_t, wv_t, wo_t, l1_t, l2_t, rw1_t, rw2_t,
      bq, bk, bv, bo, b1, b2, rb1,
      n1g, n1b, n2g, n2b)

    logits, idx = pl.pallas_call(
        _final_kernel,
        out_shape=(jax.ShapeDtypeStruct((1, E), f32),
                   jax.ShapeDtypeStruct((1, 1), jnp.int32)),
    )(sf, st, rb2, ng, nb, voter)

    idx_arr = idx.reshape((1,))
    final = pl.pallas_call(
        _gather_kernel,
        grid_spec=pltpu.PrefetchScalarGridSpec(
            num_scalar_prefetch=1,
            grid=(NIMG,),
            in_specs=[pl.BlockSpec((1, 1, L, D),
                                   lambda n, idx_ref: (idx_ref[0], n, 0, 0))],
            out_specs=pl.BlockSpec((1, L, D), lambda n, idx_ref: (n, 0, 0)),
        ),
        out_shape=jax.ShapeDtypeStruct((NIMG, L, D), f32),
    )(idx_arr, dif_granularity_features)

    return final, logits


# speedup vs baseline: 2.5480x; 1.0368x over previous
import functools

import jax
import jax.numpy as jnp
import numpy as np
from jax.experimental import pallas as pl
from jax.experimental.pallas import tpu as pltpu

D = 1024
E = 4
L = 576
NIMG = 4
T_TXT = 64
TQ = 32
NUM_TOKENS = E * L + TQ
DFF = 1024
H = 8
DH = D // H
TILE_L = 288
NLB = L // TILE_L

_F32 = jnp.float32
_BF16 = jnp.bfloat16


def _ln(x, g, b, eps=1e-5):
    mu = x.mean(-1, keepdims=True)
    var = ((x - mu) ** 2).mean(-1, keepdims=True)
    return (x - mu) * jax.lax.rsqrt(var + eps) * g + b


def _gelu(x):
    return 0.5 * x * (1.0 + jax.lax.erf(x * np.float32(1.0 / np.sqrt(2.0))))


def _dot(a, b):
    return jnp.dot(a, b, preferred_element_type=_F32,
                   precision=jax.lax.Precision.HIGHEST)


def _r(a):
    return a.astype(_BF16).astype(_F32)


def _dotb(a, b):
    return jnp.dot(a.astype(_BF16), b, preferred_element_type=_F32)


def _text_kernel(img_ref, txt_ref, mrow_ref, vt_ref,
                 wv_ref, bv_ref, wo_ref, bo_ref, l1_ref, b1_ref, l2_ref, b2_ref,
                 rw1_ref, rb1_ref, rw2_ref,
                 n1g_ref, n1b_ref, n2g_ref, n2b_ref, out_ref):
    img = img_ref[:]
    txt = txt_ref[:]
    nrm = jnp.sqrt((img * img).sum(-1, keepdims=True))
    xn = img / jnp.maximum(nrm, 1e-8)
    tnr = jnp.sqrt((txt * txt).sum(-1, keepdims=True))
    tn = txt / jnp.maximum(tnr, 1e-8)
    sim = jax.lax.dot_general(xn, tn, (((1,), (1,)), ((), ())),
                              preferred_element_type=_F32)
    neg = np.float32(-1e30)
    sim = jnp.where(mrow_ref[:] == 0.0, neg, sim)
    a_row = -sim.mean(0, keepdims=True)
    a_col = jnp.transpose(a_row)
    AR = jnp.broadcast_to(a_row, (T_TXT, T_TXT))
    AC = jnp.broadcast_to(a_col, (T_TXT, T_TXT))
    ii = jax.lax.broadcasted_iota(jnp.int32, (T_TXT, T_TXT), 0)
    jj = jax.lax.broadcasted_iota(jnp.int32, (T_TXT, T_TXT), 1)
    before = (AC < AR) | ((AC == AR) & (ii < jj))
    rank_row = before.astype(_F32).sum(0, keepdims=True)
    rr = jnp.broadcast_to(rank_row, (TQ, T_TXT))
    si = jax.lax.broadcasted_iota(jnp.int32, (TQ, T_TXT), 0).astype(_F32)
    P = (rr == si).astype(_F32)
    new_text = _dot(P, txt)
    v = _dotb(new_text, wv_ref[:]) + bv_ref[:]
    ao = _dotb(v, wo_ref[:]) + bo_ref[:]
    x1 = _ln(new_text + ao, n1g_ref[:], n1b_ref[:])
    ffh = jnp.maximum(_dotb(x1, l1_ref[:]) + b1_ref[:], 0.0)
    ff = _dotb(ffh, l2_ref[:]) + b2_ref[:]
    xo = _ln(x1 + ff, n2g_ref[:], n2b_ref[:])
    h = _gelu(_dotb(xo, rw1_ref[:]) + rb1_ref[:])
    lt = _dotb(h, rw2_ref[:])
    out_ref[:] = (_r(vt_ref[:]) * _r(lt)).sum(0, keepdims=True)


def _feat_kernel(dif_ref, voter_ref, hm_ref, hmt_ref,
                 wq_ref, wk_ref, wv_ref, wo_ref, l1_ref, l2_ref, rw1_ref,
                 rw2_ref,
                 bq_ref, bk_ref, bv_ref, bo_ref, b1_ref, b2_ref, rb1_ref,
                 n1g_ref, n1b_ref, n2g_ref, n2b_ref, out_ref):
    e = pl.program_id(0)
    lb = pl.program_id(1)
    R = NIMG * TILE_L
    x = dif_ref[0].reshape(R, D)
    iscale = np.float32(1.0 / np.sqrt(DH))
    q = _dotb(x, wq_ref[:]) + bq_ref[:]
    k = _dotb(x, wk_ref[:]) + bk_ref[:]
    v = _dotb(x, wv_ref[:]) + bv_ref[:]
    qs = [_r(q[s * TILE_L:(s + 1) * TILE_L]) for s in range(NIMG)]
    ks = [_r(k[t * TILE_L:(t + 1) * TILE_L]) for t in range(NIMG)]
    vs = [_r(v[t * TILE_L:(t + 1) * TILE_L]) for t in range(NIMG)]
    attn_parts = []
    for s in range(NIMG):
        sc = []
        for t in range(NIMG):
            prod = qs[s] * ks[t]
            hi = prod.astype(_BF16)
            lo = (prod - hi.astype(_F32)).astype(_BF16)
            ss = (jnp.dot(hi, hm_ref[:], preferred_element_type=_F32)
                  + jnp.dot(lo, hm_ref[:], preferred_element_type=_F32))
            sc.append(ss * iscale)
        mx = jnp.maximum(jnp.maximum(sc[0], sc[1]), jnp.maximum(sc[2], sc[3]))
        ex = [jnp.exp(c - mx) for c in sc]
        den = ex[0] + ex[1] + ex[2] + ex[3]
        acc = None
        for t in range(NIMG):
            w = (ex[t] / den).astype(_BF16)
            wx = jnp.dot(w, hmt_ref[:], preferred_element_type=_F32)
            acc = wx * vs[t] if acc is None else acc + wx * vs[t]
        attn_parts.append(acc)
    attn = jnp.concatenate(attn_parts, axis=0)
    ao = _dotb(attn, wo_ref[:]) + bo_ref[:]
    x1 = _ln(x + ao, n1g_ref[:], n1b_ref[:])
    ffh = jnp.maximum(_dotb(x1, l1_ref[:]) + b1_ref[:], 0.0)
    ff = _dotb(ffh, l2_ref[:]) + b2_ref[:]
    xo = _ln(x1 + ff, n2g_ref[:], n2b_ref[:])
    h = _gelu(_dotb(xo, rw1_ref[:]) + rb1_ref[:])
    lt = _dotb(h, rw2_ref[:])
    off = e * L + lb * TILE_L
    w = voter_ref[pl.ds(off, TILE_L), :]
    wfull = _r(jnp.concatenate([w, w, w, w], axis=0))
    contrib = (wfull * _r(lt)).sum(0, keepdims=True)
    first = jnp.logical_and(e == 0, lb == 0)

    @pl.when(first)
    def _():
        out_ref[:] = contrib

    @pl.when(jnp.logical_not(first))
    def _():
        out_ref[:] = out_ref[:] + contrib


def _final_kernel(sf_ref, st_ref, rb2_ref, ng_ref, nb_ref, voter_ref,
                  logits_ref, idx_ref):
    vsum = voter_ref[:].sum()
    lg = 0.25 * sf_ref[:] + st_ref[:] + vsum * rb2_ref[:]
    ln = _ln(lg, ng_ref[:], nb_ref[:])
    logits_ref[:] = ln
    m = ln.max(1, keepdims=True)
    lane = jax.lax.broadcasted_iota(jnp.int32, (1, E), 1)
    idx = jnp.where(ln == m, lane, jnp.int32(10 ** 6)).min(1, keepdims=True)
    idx_ref[:] = idx


def _gather_kernel(idx_ref, dif_ref, out_ref):
    del idx_ref
    out_ref[:] = _r(dif_ref[0])


def kernel(hidden_states, dif_granularity_features, text_hidden_states,
           text_attention_mask, router_w1, router_b1, router_w2, router_b2,
           voter, norm_g, norm_b, in_proj_w, in_proj_b, out_proj_w, out_proj_b,
           lin1_w, lin1_b, lin2_w, lin2_b, norm1_g, norm1_b, norm2_g, norm2_b):
    f32 = _F32
    bf16 = _BF16
    img = hidden_states[0]
    maskf = text_attention_mask.astype(f32)
    mrow = maskf.reshape(1, T_TXT)
    wq_t = in_proj_w[:D].T.astype(bf16)
    wk_t = in_proj_w[D:2 * D].T.astype(bf16)
    wv_t = in_proj_w[2 * D:].T.astype(bf16)
    bq = in_proj_b[:D].reshape(1, D)
    bk = in_proj_b[D:2 * D].reshape(1, D)
    bv = in_proj_b[2 * D:].reshape(1, D)
    wo_t = out_proj_w.T.astype(bf16)
    bo = out_proj_b.reshape(1, D)
    l1_t = lin1_w.T.astype(bf16)
    b1 = lin1_b.reshape(1, DFF)
    l2_t = lin2_w.T.astype(bf16)
    b2 = lin2_b.reshape(1, D)
    rw1_t = router_w1.T.astype(bf16)
    rb1 = router_b1.reshape(1, D)
    rw2_t = router_w2.T.astype(bf16)
    rb2 = router_b2.reshape(1, E)
    n1g = norm1_g.reshape(1, D)
    n1b = norm1_b.reshape(1, D)
    n2g = norm2_g.reshape(1, D)
    n2b = norm2_b.reshape(1, D)
    ng = norm_g.reshape(1, E)
    nb = norm_b.reshape(1, E)
    voter_text = voter[E * L:]

    st = pl.pallas_call(
        _text_kernel,
        out_shape=jax.ShapeDtypeStruct((1, E), f32),
    )(img, text_hidden_states, mrow, voter_text,
      wv_t, bv, wo_t, bo, l1_t, b1, l2_t, b2, rw1_t, rb1, rw2_t,
      n1g, n1b, n2g, n2b)

    hm = jnp.asarray(np.repeat(np.eye(H, dtype=np.float32), DH, axis=0))
    hmt = jnp.asarray(np.repeat(np.eye(H, dtype=np.float32), DH,
                                axis=0).T.astype(np.float32)).astype(bf16)

    full = lambda shape: pl.BlockSpec(shape, lambda e, lb: tuple(0 for _ in shape))
    sf = pl.pallas_call(
        _feat_kernel,
        grid=(E, NLB),
        in_specs=[
            pl.BlockSpec((1, NIMG, TILE_L, D), lambda e, lb: (e, 0, lb, 0)),
            full((NUM_TOKENS, 1)),
            full((D, H)), full((H, D)),
            full((D, D)), full((D, D)), full((D, D)), full((D, D)),
            full((D, DFF)), full((DFF, D)), full((D, D)), full((D, E)),
            full((1, D)), full((1, D)), full((1, D)), full((1, D)),
            full((1, DFF)), full((1, D)), full((1, D)),
            full((1, D)), full((1, D)), full((1, D)), full((1, D)),
        ],
        out_specs=pl.BlockSpec((1, E), lambda e, lb: (0, 0)),
        out_shape=jax.ShapeDtypeStruct((1, E), f32),
    )(dif_granularity_features, voter, hm, hmt,
      wq_t, wk_t, wv_t, wo_t, l1_t, l2_t, rw1_t, rw2_t,
      bq, bk, bv, bo, b1, b2, rb1,
      n1g, n1b, n2g, n2b)

    logits, idx = pl.pallas_call(
        _final_kernel,
        out_shape=(jax.ShapeDtypeStruct((1, E), f32),
                   jax.ShapeDtypeStruct((1, 1), jnp.int32)),
    )(sf, st, rb2, ng, nb, voter)

    idx_arr = idx.reshape((1,))
    final = pl.pallas_call(
        _gather_kernel,
        grid_spec=pltpu.PrefetchScalarGridSpec(
            num_scalar_prefetch=1,
            grid=(NIMG,),
            in_specs=[pl.BlockSpec((1, 1, L, D),
                                   lambda n, idx_ref: (idx_ref[0], n, 0, 0))],
            out_specs=pl.BlockSpec((1, L, D), lambda n, idx_ref: (n, 0, 0)),
        ),
        out_shape=jax.ShapeDtypeStruct((NIMG, L, D), f32),
    )(idx_arr, dif_granularity_features)

    return final, logits
